# baseline (speedup 1.0000x reference)
import functools

import jax
import jax.numpy as jnp
from jax import lax
from jax.experimental import pallas as pl
from jax.experimental.pallas import tpu as pltpu
from jax.experimental.pallas import tpu_sc as plsc

N_ROWS = 160000
N_SEG = 10000
D = 256
DH = 128
CHUNK = 80
NG = CHUNK // 16
NACC = 16
N_CHUNKS = N_ROWS // CHUNK
NS = 16
NBUF = 3
CHUNKS_PER_TILE = N_CHUNKS // NS
SEG_PER_TILE = 624
SEG_TAIL = N_SEG - NS * SEG_PER_TILE

_mesh = plsc.VectorSubcoreMesh(core_axis_name="c", subcore_axis_name="s")


def _body(feat_hbm, idx_hbm, out_hbm, *rest):
    idx_sh = rest[:NBUF]
    idx_sc = rest[NBUF:2 * NBUF]
    row_bufs = rest[2 * NBUF:3 * NBUF]
    acc_bufs = rest[3 * NBUF:4 * NBUF]
    uniq_bufs = rest[4 * NBUF:5 * NBUF]
    base_bufs = rest[5 * NBUF:6 * NBUF]
    acc_sh = rest[6 * NBUF]
    lsems = rest[6 * NBUF + 1:7 * NBUF + 1]
    ssems = rest[7 * NBUF + 1:8 * NBUF + 1]
    shift_buf = rest[8 * NBUF + 1]
    uniq_wide = rest[8 * NBUF + 2]
    rows0 = row_bufs[0]

    c = lax.axis_index("c")
    s = lax.axis_index("s")
    col0 = c * DH

    zeros16 = jnp.zeros((16,), jnp.float32)
    iota16 = lax.iota(jnp.int32, 16)

    for b in range(NBUF):
        uniq_bufs[b][...] = iota16
    shift_buf[pl.ds(0, 16)] = jnp.zeros((16,), jnp.int32)
    uniq_wide[pl.ds(0, 16)] = iota16
    uniq_wide[pl.ds(16, 16)] = iota16 + 16

    def zrow(r, carry):
        for k in range(DH // 16):
            rows0[r, pl.ds(k * 16, 16)] = zeros16
        return carry

    lax.fori_loop(0, CHUNK, zrow, 0)

    seg_base = s * SEG_PER_TILE
    full = SEG_PER_TILE // CHUNK
    rem = SEG_PER_TILE - full * CHUNK
    for j in range(full):
        pltpu.sync_copy(rows0, acc_sh.at[pl.ds(seg_base + j * CHUNK, CHUNK)])
    pltpu.sync_copy(rows0.at[pl.ds(0, rem)],
                    acc_sh.at[pl.ds(seg_base + full * CHUNK, rem)])

    @pl.when(s == 0)
    def _():
        pltpu.sync_copy(rows0.at[pl.ds(0, SEG_TAIL)],
                        acc_sh.at[pl.ds(NS * SEG_PER_TILE, SEG_TAIL)])

    plsc.subcore_barrier()

    def load_issue(i, b):
        ch = s + i * NS

        @pl.when(i < CHUNKS_PER_TILE)
        def _():
            rbase = ch * CHUNK

            @pl.when(i >= NBUF)
            def _():
                pltpu.make_async_copy(
                    acc_bufs[b], acc_sh.at[uniq_bufs[b]], ssems[b]).wait()

            pltpu.async_copy(idx_hbm.at[pl.ds(rbase, CHUNK)],
                             idx_sh[b].at[pl.ds(8, CHUNK)], lsems[b])
            pltpu.async_copy(idx_hbm.at[pl.ds(rbase, CHUNK)],
                             idx_sc[b], lsems[b])
            pltpu.async_copy(
                feat_hbm.at[pl.ds(rbase, CHUNK), pl.ds(col0, DH)],
                row_bufs[b], lsems[b])

    def reduce_scatter(j, bj):
        chj = s + j * NS

        @pl.when(jnp.logical_and(j >= 0, j < CHUNKS_PER_TILE))
        def _():
            rbase = chj * CHUNK
            pltpu.make_async_copy(idx_hbm.at[pl.ds(rbase, CHUNK)],
                                  idx_sh[bj].at[pl.ds(8, CHUNK)],
                                  lsems[bj]).wait()
            pltpu.make_async_copy(idx_hbm.at[pl.ds(rbase, CHUNK)],
                                  idx_sc[bj], lsems[bj]).wait()
            pltpu.make_async_copy(
                feat_hbm.at[pl.ds(rbase, CHUNK), pl.ds(col0, DH)],
                row_bufs[bj], lsems[bj]).wait()

            carry = jnp.int32(0)
            for g in range(NG):
                a = idx_sh[bj][pl.ds(8 + 16 * g, 16)]
                p = idx_sh[bj][pl.ds(7 + 16 * g, 16)]
                neq = jnp.where(a != p, jnp.int32(1), jnp.int32(0))
                if g == 0:
                    neq = jnp.where(iota16 == 0, jnp.int32(0), neq)
                x = neq
                for k in (1, 2, 4):
                    shift_buf[pl.ds(8, 16)] = x
                    x = x + shift_buf[pl.ds(8 - k, 16)]
                shift_buf[pl.ds(8, 16)] = x
                x = x + shift_buf[pl.ds(0, 16)]
                r = x + carry
                base_bufs[bj][pl.ds(16 * g, 16)] = r
                carry = r[15]
            nd = carry + 1

            for rr in range(NACC):
                for cb in range(DH // 16):
                    acc_bufs[bj][rr, pl.ds(cb * 16, 16)] = zeros16

            @pl.when(nd <= NACC)
            def _():
                def rb(g2, carry2):
                    rv = base_bufs[bj][pl.ds(g2 * 16, 16)]
                    av = idx_sc[bj][pl.ds(g2 * 16, 16)]
                    r0 = g2 * 16
                    for l in range(16):
                        rk = rv[l]
                        uniq_wide[pl.ds(rk, 16)] = jnp.full(
                            (16,), av[l], jnp.int32)
                        for cb in range(DH // 16):
                            plsc.addupdate(
                                acc_bufs[bj].at[rk, pl.ds(cb * 16, 16)],
                                row_bufs[bj][r0 + l, pl.ds(cb * 16, 16)])
                    return carry2

                lax.fori_loop(0, NG, rb, 0)

            uniq_bufs[bj][...] = uniq_wide[pl.ds(0, 16)]
            pltpu.async_copy(acc_bufs[bj], acc_sh.at[uniq_bufs[bj]],
                             ssems[bj], add=True)

            @pl.when(nd > NACC)
            def _():
                pltpu.sync_copy(row_bufs[bj], acc_sh.at[idx_sc[bj]],
                                add=True)

    n_steps = CHUNKS_PER_TILE + NBUF - 1
    n_super = -(-n_steps // NBUF)

    def super_body(t, carry):
        for b in range(NBUF):
            i = t * NBUF + b
            load_issue(i, b)
            reduce_scatter(i - (NBUF - 1), (b + 1) % NBUF)
        return carry

    lax.fori_loop(0, n_super, super_body, 0)

    for b in range(NBUF):
        pltpu.make_async_copy(acc_bufs[b], acc_sh.at[uniq_bufs[b]],
                              ssems[b]).wait()

    plsc.subcore_barrier()

    pltpu.sync_copy(acc_sh.at[pl.ds(seg_base, SEG_PER_TILE)],
                    out_hbm.at[pl.ds(seg_base, SEG_PER_TILE), pl.ds(col0, DH)])

    @pl.when(s == 0)
    def _():
        pltpu.sync_copy(
            acc_sh.at[pl.ds(NS * SEG_PER_TILE, SEG_TAIL)],
            out_hbm.at[pl.ds(NS * SEG_PER_TILE, SEG_TAIL), pl.ds(col0, DH)])


_seg_sum = functools.partial(
    pl.kernel,
    mesh=_mesh,
    out_type=jax.ShapeDtypeStruct((N_SEG, D), jnp.float32),
    scratch_types=(
        [pltpu.VMEM((8 + CHUNK,), jnp.int32) for _ in range(NBUF)]
        + [pltpu.VMEM((CHUNK,), jnp.int32) for _ in range(NBUF)]
        + [pltpu.VMEM((CHUNK, DH), jnp.float32) for _ in range(NBUF)]
        + [pltpu.VMEM((NACC, DH), jnp.float32) for _ in range(NBUF)]
        + [pltpu.VMEM((NACC,), jnp.int32) for _ in range(NBUF)]
        + [pltpu.VMEM((CHUNK,), jnp.int32) for _ in range(NBUF)]
        + [pltpu.VMEM_SHARED((N_SEG, DH), jnp.float32)]
        + [pltpu.SemaphoreType.DMA for _ in range(2 * NBUF)]
        + [pltpu.VMEM((24,), jnp.int32)]
        + [pltpu.VMEM((32,), jnp.int32)]
    ),
)(_body)


@jax.jit
def kernel(features, structural_indices):
    idx = structural_indices.astype(jnp.int32)
    return _seg_sum(features, idx)


# speedup vs baseline: 3.0765x; 3.0765x over previous
import functools

import jax
import jax.numpy as jnp
from jax import lax
from jax.experimental import pallas as pl
from jax.experimental.pallas import tpu as pltpu
from jax.experimental.pallas import tpu_sc as plsc

N_ROWS = 160000
N_SEG = 10000
D = 256
DH = 128
CHUNK = 64
N_CHUNKS = N_ROWS // CHUNK
NS = 16
NBUF = 6
CHUNKS_PER_TILE = -(-N_CHUNKS // NS)
SEG_PER_TILE = 624
SEG_TAIL = N_SEG - NS * SEG_PER_TILE

_mesh = plsc.VectorSubcoreMesh(core_axis_name="c", subcore_axis_name="s")


def _body(feat_hbm, idx_hbm, out_hbm, *rest):
    idx_bufs = rest[:NBUF]
    row_bufs = rest[NBUF:2 * NBUF]
    acc_sh = rest[2 * NBUF]
    lsems = rest[2 * NBUF + 1:2 * NBUF + 1 + NBUF]
    ssems = rest[2 * NBUF + 1 + NBUF:]
    rows0 = row_bufs[0]

    c = lax.axis_index("c")
    s = lax.axis_index("s")
    col0 = c * DH

    zeros16 = jnp.zeros((16,), jnp.float32)

    def zrow(r, carry):
        for k in range(DH // 16):
            rows0[r, pl.ds(k * 16, 16)] = zeros16
        return carry

    lax.fori_loop(0, CHUNK, zrow, 0)

    seg_base = s * SEG_PER_TILE
    full = SEG_PER_TILE // CHUNK
    rem = SEG_PER_TILE - full * CHUNK
    for j in range(full):
        pltpu.sync_copy(rows0, acc_sh.at[pl.ds(seg_base + j * CHUNK, CHUNK)])
    pltpu.sync_copy(rows0.at[pl.ds(0, rem)],
                    acc_sh.at[pl.ds(seg_base + full * CHUNK, rem)])

    @pl.when(s == 0)
    def _():
        pltpu.sync_copy(rows0.at[pl.ds(0, SEG_TAIL)],
                        acc_sh.at[pl.ds(NS * SEG_PER_TILE, SEG_TAIL)])

    plsc.subcore_barrier()

    def load_issue(i, b):
        ch = s + i * NS

        @pl.when(ch < N_CHUNKS)
        def _():
            rbase = ch * CHUNK

            @pl.when(i >= NBUF)
            def _():
                pltpu.make_async_copy(
                    row_bufs[b], acc_sh.at[idx_bufs[b]], ssems[b]).wait()

            pltpu.async_copy(idx_hbm.at[pl.ds(rbase, CHUNK)],
                             idx_bufs[b], lsems[b])
            pltpu.async_copy(
                feat_hbm.at[pl.ds(rbase, CHUNK), pl.ds(col0, DH)],
                row_bufs[b], lsems[b])

    def scatter_issue(j, bj):
        chj = s + j * NS

        @pl.when(jnp.logical_and(j >= 0, chj < N_CHUNKS))
        def _():
            rbase = chj * CHUNK
            pltpu.make_async_copy(idx_hbm.at[pl.ds(rbase, CHUNK)],
                                  idx_bufs[bj], lsems[bj]).wait()
            pltpu.make_async_copy(
                feat_hbm.at[pl.ds(rbase, CHUNK), pl.ds(col0, DH)],
                row_bufs[bj], lsems[bj]).wait()
            pltpu.async_copy(row_bufs[bj], acc_sh.at[idx_bufs[bj]],
                             ssems[bj], add=True)

    n_steps = CHUNKS_PER_TILE + NBUF - 1
    n_super = -(-n_steps // NBUF)

    def super_body(t, carry):
        for b in range(NBUF):
            i = t * NBUF + b
            load_issue(i, b)
            scatter_issue(i - (NBUF - 1), (b + 1) % NBUF)
        return carry

    lax.fori_loop(0, n_super, super_body, 0)

    for b in range(NBUF):
        pltpu.make_async_copy(row_bufs[b], acc_sh.at[idx_bufs[b]],
                              ssems[b]).wait()

    plsc.subcore_barrier()

    pltpu.sync_copy(acc_sh.at[pl.ds(seg_base, SEG_PER_TILE)],
                    out_hbm.at[pl.ds(seg_base, SEG_PER_TILE), pl.ds(col0, DH)])

    @pl.when(s == 0)
    def _():
        pltpu.sync_copy(
            acc_sh.at[pl.ds(NS * SEG_PER_TILE, SEG_TAIL)],
            out_hbm.at[pl.ds(NS * SEG_PER_TILE, SEG_TAIL), pl.ds(col0, DH)])


_seg_sum = functools.partial(
    pl.kernel,
    mesh=_mesh,
    out_type=jax.ShapeDtypeStruct((N_SEG, D), jnp.float32),
    scratch_types=(
        [pltpu.VMEM((CHUNK,), jnp.int32) for _ in range(NBUF)]
        + [pltpu.VMEM((CHUNK, DH), jnp.float32) for _ in range(NBUF)]
        + [pltpu.VMEM_SHARED((N_SEG, DH), jnp.float32)]
        + [pltpu.SemaphoreType.DMA for _ in range(2 * NBUF)]
    ),
)(_body)


@jax.jit
def kernel(features, structural_indices):
    idx = structural_indices.astype(jnp.int32)
    return _seg_sum(features, idx)
